# baseline (speedup 1.0000x reference)
import functools

import jax
import jax.numpy as jnp
from jax import lax
from jax.experimental import pallas as pl
from jax.experimental.pallas import tpu as pltpu
from jax.experimental.pallas import tpu_sc as plsc

_ROWS_PER_BLOCK = 4096
_LANES = 16


def _sc_gather_body(idx_hbm, sites_hbm, masks_hbm, site_out, mask_out,
                    idx_v, sv, mv, sem):
    c = lax.axis_index("c")
    s = lax.axis_index("s")

    @pl.when((c == 0) & (s == 0))
    def _():
        pltpu.sync_copy(idx_hbm, idx_v)
        pltpu.async_copy(sites_hbm.at[idx_v], sv, sem).wait()
        pltpu.async_copy(masks_hbm.at[idx_v], mv, sem).wait()
        pltpu.sync_copy(sv, site_out)
        pltpu.sync_copy(mv, mask_out)


def _sc_gather(idx, sites16, masks16):
    B = idx.shape[0]
    mesh = plsc.VectorSubcoreMesh(core_axis_name="c", subcore_axis_name="s")
    kern = functools.partial(
        pl.kernel,
        mesh=mesh,
        out_type=[
            jax.ShapeDtypeStruct((B,), jnp.float32),
            jax.ShapeDtypeStruct((B,), jnp.float32),
        ],
        scratch_types=[
            pltpu.VMEM((B,), jnp.int32),
            pltpu.VMEM((B,), jnp.float32),
            pltpu.VMEM((B,), jnp.float32),
            pltpu.SemaphoreType.DMA,
        ],
    )(_sc_gather_body)
    return kern(idx, sites16, masks16)


def _fma_body(rows_per_b, site_ref, mask_ref, x_ref, o_ref):
    rb = x_ref.shape[0]
    if rb <= rows_per_b:
        b = (pl.program_id(0) * rb) // rows_per_b
        o_ref[...] = x_ref[...] * mask_ref[b] + site_ref[b]
    else:
        nb = rb // rows_per_b
        b0 = pl.program_id(0) * nb
        for k in range(nb):
            sl = pl.ds(k * rows_per_b, rows_per_b)
            o_ref[sl, :] = x_ref[sl, :] * mask_ref[b0 + k] + site_ref[b0 + k]


def kernel(inputs, available_injection_sites, masks):
    B, H, W, C = inputs.shape
    n = available_injection_sites.shape[0]
    idx = jax.random.randint(jax.random.key(22), (B,), 0, n).astype(jnp.int32)
    sites16 = jnp.zeros((_LANES,), jnp.float32).at[:n].set(
        available_injection_sites.reshape(n))
    masks16 = jnp.zeros((_LANES,), jnp.float32).at[:n].set(masks.reshape(n))

    site_b, mask_b = _sc_gather(idx, sites16, masks16)

    rows_per_b = H * W
    rb = _ROWS_PER_BLOCK
    if (B * rows_per_b) % rb or (rb % rows_per_b and rows_per_b % rb):
        rb = rows_per_b
    total = B * rows_per_b
    x = inputs.reshape(total, C)

    out = pl.pallas_call(
        functools.partial(_fma_body, rows_per_b),
        grid=(total // rb,),
        in_specs=[
            pl.BlockSpec(memory_space=pltpu.SMEM),
            pl.BlockSpec(memory_space=pltpu.SMEM),
            pl.BlockSpec((rb, C), lambda i: (i, 0)),
        ],
        out_specs=pl.BlockSpec((rb, C), lambda i: (i, 0)),
        out_shape=jax.ShapeDtypeStruct((total, C), inputs.dtype),
        compiler_params=pltpu.CompilerParams(
            dimension_semantics=("parallel",),
        ),
    )(site_b, mask_b, x)
    return out.reshape(B, H, W, C)


# speedup vs baseline: 1.2453x; 1.2453x over previous
import jax
import jax.numpy as jnp
from jax.experimental import pallas as pl
from jax.experimental.pallas import tpu as pltpu

_ROWS_PER_BLOCK = 4096


def _fma_body(rows_per_b, idx_ref, site_ref, mask_ref, x_ref, o_ref):
    rb = x_ref.shape[0]
    if rb <= rows_per_b:
        b = (pl.program_id(0) * rb) // rows_per_b
        j = idx_ref[b]
        o_ref[...] = x_ref[...] * mask_ref[j] + site_ref[j]
    else:
        nb = rb // rows_per_b
        b0 = pl.program_id(0) * nb
        for k in range(nb):
            j = idx_ref[b0 + k]
            sl = pl.ds(k * rows_per_b, rows_per_b)
            o_ref[sl, :] = x_ref[sl, :] * mask_ref[j] + site_ref[j]


def kernel(inputs, available_injection_sites, masks):
    B, H, W, C = inputs.shape
    n = available_injection_sites.shape[0]
    idx = jax.random.randint(jax.random.key(22), (B,), 0, n).astype(jnp.int32)
    sites = available_injection_sites.reshape(n)
    msk = masks.reshape(n)

    rows_per_b = H * W
    rb = _ROWS_PER_BLOCK
    if rows_per_b % rb:
        rb = rows_per_b
    total = B * rows_per_b
    x = inputs.reshape(total, C)

    import functools
    out = pl.pallas_call(
        functools.partial(_fma_body, rows_per_b),
        grid=(total // rb,),
        in_specs=[
            pl.BlockSpec(memory_space=pltpu.SMEM),
            pl.BlockSpec(memory_space=pltpu.SMEM),
            pl.BlockSpec(memory_space=pltpu.SMEM),
            pl.BlockSpec((rb, C), lambda i: (i, 0)),
        ],
        out_specs=pl.BlockSpec((rb, C), lambda i: (i, 0)),
        out_shape=jax.ShapeDtypeStruct((total, C), inputs.dtype),
        compiler_params=pltpu.CompilerParams(
            dimension_semantics=("parallel",),
        ),
    )(idx, sites, msk, x)
    return out.reshape(B, H, W, C)


# speedup vs baseline: 1.3115x; 1.0532x over previous
import jax
import jax.numpy as jnp
from jax.experimental import pallas as pl
from jax.experimental.pallas import tpu as pltpu

_ROWS_PER_BLOCK = 4096


def _fma_body(rows_per_b, idx_ref, site_ref, mask_ref, x_ref, o_ref):
    rb = x_ref.shape[0]
    if rb <= rows_per_b:
        b = (pl.program_id(0) * rb) // rows_per_b
        j = idx_ref[b]
        o_ref[...] = x_ref[...] * mask_ref[j] + site_ref[j]
    else:
        nb = rb // rows_per_b
        b0 = pl.program_id(0) * nb
        for k in range(nb):
            j = idx_ref[b0 + k]
            sl = pl.ds(k * rows_per_b, rows_per_b)
            o_ref[sl, :] = x_ref[sl, :] * mask_ref[j] + site_ref[j]


def kernel(inputs, available_injection_sites, masks):
    B, H, W, C = inputs.shape
    n = available_injection_sites.shape[0]
    idx = jax.random.randint(jax.random.key(22), (B,), 0, n).astype(jnp.int32)
    sites = available_injection_sites.reshape(n)
    msk = masks.reshape(n)

    rows_per_b = H * W
    rb = _ROWS_PER_BLOCK
    if not (rb % rows_per_b == 0 or rows_per_b % rb == 0):
        rb = rows_per_b
    total = B * rows_per_b
    x = inputs.reshape(total, C)

    import functools
    out = pl.pallas_call(
        functools.partial(_fma_body, rows_per_b),
        grid=(total // rb,),
        in_specs=[
            pl.BlockSpec(memory_space=pltpu.SMEM),
            pl.BlockSpec(memory_space=pltpu.SMEM),
            pl.BlockSpec(memory_space=pltpu.SMEM),
            pl.BlockSpec((rb, C), lambda i: (i, 0)),
        ],
        out_specs=pl.BlockSpec((rb, C), lambda i: (i, 0)),
        out_shape=jax.ShapeDtypeStruct((total, C), inputs.dtype),
        compiler_params=pltpu.CompilerParams(
            dimension_semantics=("parallel",),
        ),
    )(idx, sites, msk, x)
    return out.reshape(B, H, W, C)
